# Initial kernel scaffold; baseline (speedup 1.0000x reference)
import jax
import jax.numpy as jnp
from jax.experimental import pallas as pl


def kernel(features, bank_features, bank_probs, image_bank):
    raise NotImplementedError("write your pallas kernel here")


# speedup vs baseline: 2.7004x; 2.7004x over previous
import functools

import jax
import jax.numpy as jnp
from jax import lax
from jax.experimental import pallas as pl
from jax.experimental.pallas import tpu as pltpu
from jax.experimental.pallas import tpu_sc as plsc

Q = 256
D = 512
K = 16384
KNN = 4
C_PAD = 128
IMG = 3072
KB = 2048
NSTEPS = K // KB
CHUNK = 512
NCHUNK = K // CHUNK
EPS = 1e-12
BIGI = 2**30


def _dist_topk_body(f_ref, b_ref, idx_ref, qn_ref, d_ref):
    step = pl.program_id(0)

    @pl.when(step == 0)
    def _():
        f = f_ref[:]
        nrm = jnp.sqrt(jnp.sum(f * f, axis=1, keepdims=True))
        qn_ref[:] = f / jnp.maximum(nrm, EPS)

    b = b_ref[:]
    nrm = jnp.sqrt(jnp.sum(b * b, axis=1, keepdims=True))
    kn = b / jnp.maximum(nrm, EPS)
    m = lax.dot_general(qn_ref[:], kn, (((1,), (1,)), ((), ())),
                        preferred_element_type=jnp.float32,
                        precision=lax.Precision.DEFAULT)
    dist = 1.0 - m
    for t in range(KB // CHUNK):
        d_ref[step * (KB // CHUNK) + t] = dist[:, t * CHUNK:(t + 1) * CHUNK]

    @pl.when(step == NSTEPS - 1)
    def _():
        found = []
        for j in range(KNN):
            def scan_chunk(c, carry):
                runmax, runidx = carry
                blk = d_ref[c]
                col = lax.broadcasted_iota(jnp.int32, (Q, CHUNK), 1) + c * CHUNK
                for f in found:
                    blk = jnp.where(col == f, -jnp.inf, blk)
                mc = jnp.max(blk, axis=1, keepdims=True)
                ac = jnp.min(jnp.where(blk == mc, col, BIGI),
                             axis=1, keepdims=True)
                take = mc > runmax
                return (jnp.where(take, mc, runmax),
                        jnp.where(take, ac, runidx))
            init = (jnp.full((Q, 1), -jnp.inf, jnp.float32),
                    jnp.full((Q, 1), BIGI, jnp.int32))
            _, best = lax.fori_loop(0, NCHUNK, scan_chunk, init)
            idx_ref[:, j:j + 1] = best
            found.append(best)


def _dist_topk(features, bank_features):
    return pl.pallas_call(
        _dist_topk_body,
        grid=(NSTEPS,),
        in_specs=[
            pl.BlockSpec((Q, D), lambda i: (0, 0)),
            pl.BlockSpec((KB, D), lambda i: (i, 0)),
        ],
        out_specs=pl.BlockSpec((Q, KNN), lambda i: (0, 0)),
        out_shape=jax.ShapeDtypeStruct((Q, KNN), jnp.int32),
        scratch_shapes=[
            pltpu.VMEM((Q, D), jnp.float32),
            pltpu.VMEM((NCHUNK, Q, CHUNK), jnp.float32),
        ],
    )(features, bank_features)


QPW = Q // 32


def _sc_gather_body(idx_hbm, feat_hbm, probs_hbm, img_hbm,
                    grads_out, probs_out, imgs_out,
                    idx_v, feat_v, probs_v, img_v, gsum_v, psum_v, isum_v,
                    sem):
    nc = 2
    wid = lax.axis_index("c") * 16 + lax.axis_index("s")
    del nc
    qbase = wid * QPW

    pltpu.sync_copy(idx_hbm.at[pl.ds(qbase * KNN, QPW * KNN)], idx_v)
    pltpu.async_copy(feat_hbm.at[idx_v], feat_v, sem).wait()
    pltpu.async_copy(probs_hbm.at[idx_v], probs_v, sem).wait()

    for q in range(QPW):
        def fbody(i, _, q=q):
            s = pl.ds(i * 16, 16)
            gsum_v[q, s] = 0.25 * ((feat_v[4 * q + 0, s] + feat_v[4 * q + 1, s])
                                   + (feat_v[4 * q + 2, s] + feat_v[4 * q + 3, s]))
            return 0
        lax.fori_loop(0, D // 16, fbody, 0)

        def pbody(i, _, q=q):
            s = pl.ds(i * 16, 16)
            psum_v[q, s] = 0.25 * ((probs_v[4 * q + 0, s] + probs_v[4 * q + 1, s])
                                   + (probs_v[4 * q + 2, s] + probs_v[4 * q + 3, s]))
            return 0
        lax.fori_loop(0, C_PAD // 16, pbody, 0)
    pltpu.sync_copy(gsum_v, grads_out.at[pl.ds(qbase, QPW)])
    pltpu.sync_copy(psum_v, probs_out.at[pl.ds(qbase, QPW)])

    for p in range(QPW // 2):
        pltpu.async_copy(img_hbm.at[idx_v.at[pl.ds(p * 8, 8)]], img_v, sem).wait()
        for t in range(2):
            def ibody(i, _, t=t):
                s = pl.ds(i * 16, 16)
                isum_v[t, s] = 0.25 * ((img_v[4 * t + 0, s] + img_v[4 * t + 1, s])
                                       + (img_v[4 * t + 2, s] + img_v[4 * t + 3, s]))
                return 0
            lax.fori_loop(0, IMG // 16, ibody, 0)
        pltpu.sync_copy(isum_v, imgs_out.at[pl.ds(qbase + p * 2, 2)])


def _sc_gather(idx_flat, bank_features, probs_pad, img_flat):
    mesh = plsc.VectorSubcoreMesh(core_axis_name="c", subcore_axis_name="s")
    fn = pl.kernel(
        _sc_gather_body,
        out_type=[
            jax.ShapeDtypeStruct((Q, D), jnp.float32),
            jax.ShapeDtypeStruct((Q, C_PAD), jnp.float32),
            jax.ShapeDtypeStruct((Q, IMG), jnp.float32),
        ],
        mesh=mesh,
        scratch_types=[
            pltpu.VMEM((QPW * KNN,), jnp.int32),
            pltpu.VMEM((QPW * KNN, D), jnp.float32),
            pltpu.VMEM((QPW * KNN, C_PAD), jnp.float32),
            pltpu.VMEM((8, IMG), jnp.float32),
            pltpu.VMEM((QPW, D), jnp.float32),
            pltpu.VMEM((QPW, C_PAD), jnp.float32),
            pltpu.VMEM((2, IMG), jnp.float32),
            pltpu.SemaphoreType.DMA,
        ],
    )
    return fn(idx_flat, bank_features, probs_pad, img_flat)


def _argmax_body(p_ref, out_ref):
    p = p_ref[:]
    col = lax.broadcasted_iota(jnp.int32, (Q, C_PAD), 1)
    pm = jnp.where(col < 10, p, -jnp.inf)
    m = jnp.max(pm, axis=1, keepdims=True)
    out_ref[:] = jnp.min(jnp.where(pm == m, col, BIGI), axis=1, keepdims=True)


def _argmax(probs16):
    return pl.pallas_call(
        _argmax_body,
        out_shape=jax.ShapeDtypeStruct((Q, 1), jnp.int32),
    )(probs16)


def kernel(features, bank_features, bank_probs, image_bank):
    probs_pad = jnp.pad(bank_probs, ((0, 0), (0, C_PAD - bank_probs.shape[1])))
    img_flat = image_bank.reshape(K, IMG)

    idx = _dist_topk(features, bank_features)
    idx_flat = idx.reshape(Q * KNN)
    grads, probs16, imgs = _sc_gather(idx_flat, bank_features, probs_pad, img_flat)
    labels = _argmax(probs16).reshape(Q)

    probs = probs16[:, :10]
    images = imgs.reshape(Q, 3, 32, 32)
    return (labels, probs, images, grads)


# speedup vs baseline: 4.7318x; 1.7523x over previous
import jax
import jax.numpy as jnp
from jax import lax
from jax.experimental import pallas as pl
from jax.experimental.pallas import tpu as pltpu
from jax.experimental.pallas import tpu_sc as plsc

Q = 256
D = 512
K = 16384
KNN = 4
NCLS = 10
IMG = 3072
KB = 2048
NSTEPS = K // KB
CHUNK = 512
NCHUNK = K // CHUNK
MB = 768
NMB = IMG // MB
EPS = 1e-12
BIGI = 2**30


def _dist_topk_body(f_ref, b_ref, idx_ref, w_ref, qn_ref, d_ref):
    step = pl.program_id(0)

    @pl.when(step == 0)
    def _():
        f = f_ref[:]
        nrm = jnp.sqrt(jnp.sum(f * f, axis=1, keepdims=True))
        qn_ref[:] = f / jnp.maximum(nrm, EPS)

    b = b_ref[:]
    nrm = jnp.sqrt(jnp.sum(b * b, axis=1, keepdims=True))
    kn = b / jnp.maximum(nrm, EPS)
    m = lax.dot_general(qn_ref[:], kn, (((1,), (1,)), ((), ())),
                        preferred_element_type=jnp.float32,
                        precision=lax.Precision.DEFAULT)
    dist = 1.0 - m
    for t in range(KB // CHUNK):
        d_ref[step * (KB // CHUNK) + t] = dist[:, t * CHUNK:(t + 1) * CHUNK]

    @pl.when(step == NSTEPS - 1)
    def _():
        found = []
        for j in range(KNN):
            def scan_chunk(c, carry):
                runmax, runidx = carry
                blk = d_ref[c]
                col = lax.broadcasted_iota(jnp.int32, (Q, CHUNK), 1) + c * CHUNK
                for f in found:
                    blk = jnp.where(col == f, -jnp.inf, blk)
                mc = jnp.max(blk, axis=1, keepdims=True)
                ac = jnp.min(jnp.where(blk == mc, col, BIGI),
                             axis=1, keepdims=True)
                take = mc > runmax
                return (jnp.where(take, mc, runmax),
                        jnp.where(take, ac, runidx))
            init = (jnp.full((Q, 1), -jnp.inf, jnp.float32),
                    jnp.full((Q, 1), BIGI, jnp.int32))
            _, best = lax.fori_loop(0, NCHUNK, scan_chunk, init)
            idx_ref[:, j:j + 1] = best
            found.append(best)
        for c in range(NCHUNK):
            col = lax.broadcasted_iota(jnp.int32, (Q, CHUNK), 1) + c * CHUNK
            w = jnp.zeros((Q, CHUNK), jnp.float32)
            for f in found:
                w = jnp.where(col == f, 0.25, w)
            w_ref[:, c * CHUNK:(c + 1) * CHUNK] = w.astype(jnp.bfloat16)


def _dist_topk(features, bank_features):
    return pl.pallas_call(
        _dist_topk_body,
        grid=(NSTEPS,),
        in_specs=[
            pl.BlockSpec((Q, D), lambda i: (0, 0)),
            pl.BlockSpec((KB, D), lambda i: (i, 0)),
        ],
        out_specs=[
            pl.BlockSpec((Q, KNN), lambda i: (0, 0)),
            pl.BlockSpec((Q, K), lambda i: (0, 0)),
        ],
        out_shape=[
            jax.ShapeDtypeStruct((Q, KNN), jnp.int32),
            jax.ShapeDtypeStruct((Q, K), jnp.bfloat16),
        ],
        scratch_shapes=[
            pltpu.VMEM((Q, D), jnp.float32),
            pltpu.VMEM((NCHUNK, Q, CHUNK), jnp.float32),
        ],
    )(features, bank_features)


QPW = Q // 32


def _sc_gather_body(idx_hbm, feat_hbm, grads_out, idx_v, feat_v, gsum_v, sem):
    wid = lax.axis_index("c") * 16 + lax.axis_index("s")
    qbase = wid * QPW

    pltpu.sync_copy(idx_hbm.at[pl.ds(qbase * KNN, QPW * KNN)], idx_v)
    pltpu.async_copy(feat_hbm.at[idx_v], feat_v, sem).wait()

    for q in range(QPW):
        def fbody(i, _, q=q):
            s = pl.ds(i * 16, 16)
            gsum_v[q, s] = 0.25 * ((feat_v[4 * q + 0, s] + feat_v[4 * q + 1, s])
                                   + (feat_v[4 * q + 2, s] + feat_v[4 * q + 3, s]))
            return 0
        lax.fori_loop(0, D // 16, fbody, 0)
    pltpu.sync_copy(gsum_v, grads_out.at[pl.ds(qbase, QPW)])


def _sc_gather(idx_flat, bank_features):
    mesh = plsc.VectorSubcoreMesh(core_axis_name="c", subcore_axis_name="s")
    fn = pl.kernel(
        _sc_gather_body,
        out_type=jax.ShapeDtypeStruct((Q, D), jnp.float32),
        mesh=mesh,
        scratch_types=[
            pltpu.VMEM((QPW * KNN,), jnp.int32),
            pltpu.VMEM((QPW * KNN, D), jnp.float32),
            pltpu.VMEM((QPW, D), jnp.float32),
            pltpu.SemaphoreType.DMA,
        ],
    )
    return fn(idx_flat, bank_features)


def _select_body(img_ref, w_ref, p_ref, imgs_ref, probs_ref, lab_ref):
    k = pl.program_id(0)
    m = pl.program_id(1)
    w = w_ref[:].astype(jnp.float32)
    blk = lax.dot_general(img_ref[:], w, (((1,), (1,)), ((), ())),
                          preferred_element_type=jnp.float32,
                          precision=lax.Precision.DEFAULT)
    sl = pl.ds(m * MB, MB)

    @pl.when(k == 0)
    def _():
        imgs_ref[sl, :] = blk

    @pl.when(k > 0)
    def _():
        imgs_ref[sl, :] += blk

    @pl.when(m == 0)
    def _():
        pblk = lax.dot_general(p_ref[:], w, (((1,), (1,)), ((), ())),
                               preferred_element_type=jnp.float32,
                               precision=lax.Precision.HIGHEST)

        @pl.when(k == 0)
        def _():
            probs_ref[:] = pblk

        @pl.when(k > 0)
        def _():
            probs_ref[:] += pblk

    @pl.when((m == NMB - 1) & (k == NSTEPS - 1))
    def _():
        p = probs_ref[:]
        row = lax.broadcasted_iota(jnp.int32, (NCLS, Q), 0)
        top = jnp.max(p, axis=0, keepdims=True)
        lab_ref[:] = jnp.min(jnp.where(p == top, row, BIGI),
                             axis=0, keepdims=True)


def _select(img_t, w, probs_t):
    return pl.pallas_call(
        _select_body,
        grid=(NSTEPS, NMB),
        in_specs=[
            pl.BlockSpec((MB, KB), lambda k, m: (m, k)),
            pl.BlockSpec((Q, KB), lambda k, m: (0, k)),
            pl.BlockSpec((NCLS, KB), lambda k, m: (0, k)),
        ],
        out_specs=[
            pl.BlockSpec((IMG, Q), lambda k, m: (0, 0)),
            pl.BlockSpec((NCLS, Q), lambda k, m: (0, 0)),
            pl.BlockSpec((1, Q), lambda k, m: (0, 0)),
        ],
        out_shape=[
            jax.ShapeDtypeStruct((IMG, Q), jnp.float32),
            jax.ShapeDtypeStruct((NCLS, Q), jnp.float32),
            jax.ShapeDtypeStruct((1, Q), jnp.int32),
        ],
    )(img_t, w, probs_t)


def kernel(features, bank_features, bank_probs, image_bank):
    img_t = image_bank.transpose(1, 2, 3, 0).reshape(IMG, K)
    probs_t = bank_probs.transpose(1, 0)

    idx, w = _dist_topk(features, bank_features)
    grads = _sc_gather(idx.reshape(Q * KNN), bank_features)
    imgs_t, probs_t_out, labels = _select(img_t, w, probs_t)

    images = imgs_t.reshape(3, 32, 32, Q).transpose(3, 0, 1, 2)
    probs = probs_t_out.transpose(1, 0)
    return (labels.reshape(Q), probs, images, grads)


# speedup vs baseline: 4.8995x; 1.0354x over previous
import jax
import jax.numpy as jnp
from jax import lax
from jax.experimental import pallas as pl
from jax.experimental.pallas import tpu as pltpu
from jax.experimental.pallas import tpu_sc as plsc

Q = 256
D = 512
K = 16384
KNN = 4
NCLS = 10
IMG = 3072
KB = 2048
NSTEPS = K // KB
CHUNK = 512
NCHUNK = K // CHUNK
KBS = 4096
NKS = K // KBS
MB = 1024
NMB = IMG // MB
EPS = 1e-12
BIGI = 2**30


def _dist_topk_body(f_ref, b_ref, idx_ref, w_ref, qn_ref, d_ref):
    step = pl.program_id(0)

    @pl.when(step == 0)
    def _():
        f = f_ref[:]
        nrm = jnp.sqrt(jnp.sum(f * f, axis=1, keepdims=True))
        qn_ref[:] = f / jnp.maximum(nrm, EPS)

    b = b_ref[:]
    nrm = jnp.sqrt(jnp.sum(b * b, axis=1, keepdims=True))
    kn = b / jnp.maximum(nrm, EPS)
    m = lax.dot_general(qn_ref[:], kn, (((1,), (1,)), ((), ())),
                        preferred_element_type=jnp.float32,
                        precision=lax.Precision.DEFAULT)
    dist = 1.0 - m
    for t in range(KB // CHUNK):
        d_ref[step * (KB // CHUNK) + t] = dist[:, t * CHUNK:(t + 1) * CHUNK]

    @pl.when(step == NSTEPS - 1)
    def _():
        found = []
        for j in range(KNN):
            def scan_chunk(c, carry):
                runmax, runidx = carry
                blk = d_ref[c]
                col = lax.broadcasted_iota(jnp.int32, (Q, CHUNK), 1) + c * CHUNK
                for f in found:
                    blk = jnp.where(col == f, -jnp.inf, blk)
                mc = jnp.max(blk, axis=1, keepdims=True)
                ac = jnp.min(jnp.where(blk == mc, col, BIGI),
                             axis=1, keepdims=True)
                take = mc > runmax
                return (jnp.where(take, mc, runmax),
                        jnp.where(take, ac, runidx))
            init = (jnp.full((Q, 1), -jnp.inf, jnp.float32),
                    jnp.full((Q, 1), BIGI, jnp.int32))
            _, best = lax.fori_loop(0, NCHUNK, scan_chunk, init)
            idx_ref[:, j:j + 1] = best
            found.append(best)
        for c in range(NCHUNK):
            col = lax.broadcasted_iota(jnp.int32, (Q, CHUNK), 1) + c * CHUNK
            w = jnp.zeros((Q, CHUNK), jnp.float32)
            for f in found:
                w = jnp.where(col == f, 0.25, w)
            w_ref[:, c * CHUNK:(c + 1) * CHUNK] = w.astype(jnp.bfloat16)


def _dist_topk(features, bank_features):
    return pl.pallas_call(
        _dist_topk_body,
        grid=(NSTEPS,),
        in_specs=[
            pl.BlockSpec((Q, D), lambda i: (0, 0)),
            pl.BlockSpec((KB, D), lambda i: (i, 0)),
        ],
        out_specs=[
            pl.BlockSpec((Q, KNN), lambda i: (0, 0)),
            pl.BlockSpec((Q, K), lambda i: (0, 0)),
        ],
        out_shape=[
            jax.ShapeDtypeStruct((Q, KNN), jnp.int32),
            jax.ShapeDtypeStruct((Q, K), jnp.bfloat16),
        ],
        scratch_shapes=[
            pltpu.VMEM((Q, D), jnp.float32),
            pltpu.VMEM((NCHUNK, Q, CHUNK), jnp.float32),
        ],
    )(features, bank_features)


QPW = Q // 32


def _sc_gather_body(idx_hbm, feat_hbm, grads_out, idx_v, feat_v, gsum_v, sem):
    wid = lax.axis_index("c") * 16 + lax.axis_index("s")
    qbase = wid * QPW

    pltpu.sync_copy(idx_hbm.at[pl.ds(qbase * KNN, QPW * KNN)], idx_v)
    pltpu.async_copy(feat_hbm.at[idx_v], feat_v, sem).wait()

    for q in range(QPW):
        def fbody(i, _, q=q):
            s = pl.ds(i * 16, 16)
            gsum_v[q, s] = 0.25 * ((feat_v[4 * q + 0, s] + feat_v[4 * q + 1, s])
                                   + (feat_v[4 * q + 2, s] + feat_v[4 * q + 3, s]))
            return 0
        lax.fori_loop(0, D // 16, fbody, 0)
    pltpu.sync_copy(gsum_v, grads_out.at[pl.ds(qbase, QPW)])


def _sc_gather(idx_flat, bank_features):
    mesh = plsc.VectorSubcoreMesh(core_axis_name="c", subcore_axis_name="s")
    fn = pl.kernel(
        _sc_gather_body,
        out_type=jax.ShapeDtypeStruct((Q, D), jnp.float32),
        mesh=mesh,
        scratch_types=[
            pltpu.VMEM((QPW * KNN,), jnp.int32),
            pltpu.VMEM((QPW * KNN, D), jnp.float32),
            pltpu.VMEM((QPW, D), jnp.float32),
            pltpu.SemaphoreType.DMA,
        ],
    )
    return fn(idx_flat, bank_features)


def _select_body(img_ref, w_ref, p_ref, imgs_ref, probs_ref, lab_ref):
    k = pl.program_id(0)
    m = pl.program_id(1)
    w = w_ref[:].astype(jnp.float32)
    blk = lax.dot_general(img_ref[:], w, (((1,), (1,)), ((), ())),
                          preferred_element_type=jnp.float32,
                          precision=lax.Precision.DEFAULT)
    sl = pl.ds(m * MB, MB)

    @pl.when(k == 0)
    def _():
        imgs_ref[sl, :] = blk

    @pl.when(k > 0)
    def _():
        imgs_ref[sl, :] += blk

    @pl.when(m == 0)
    def _():
        pblk = lax.dot_general(p_ref[:], w, (((1,), (1,)), ((), ())),
                               preferred_element_type=jnp.float32,
                               precision=lax.Precision.HIGHEST)

        @pl.when(k == 0)
        def _():
            probs_ref[:] = pblk

        @pl.when(k > 0)
        def _():
            probs_ref[:] += pblk

    @pl.when((m == NMB - 1) & (k == NKS - 1))
    def _():
        p = probs_ref[:]
        row = lax.broadcasted_iota(jnp.int32, (NCLS, Q), 0)
        top = jnp.max(p, axis=0, keepdims=True)
        lab_ref[:] = jnp.min(jnp.where(p == top, row, BIGI),
                             axis=0, keepdims=True)


def _select(img_t, w, probs_t):
    return pl.pallas_call(
        _select_body,
        grid=(NKS, NMB),
        in_specs=[
            pl.BlockSpec((MB, KBS), lambda k, m: (m, k)),
            pl.BlockSpec((Q, KBS), lambda k, m: (0, k)),
            pl.BlockSpec((NCLS, KBS), lambda k, m: (0, k)),
        ],
        out_specs=[
            pl.BlockSpec((IMG, Q), lambda k, m: (0, 0)),
            pl.BlockSpec((NCLS, Q), lambda k, m: (0, 0)),
            pl.BlockSpec((1, Q), lambda k, m: (0, 0)),
        ],
        out_shape=[
            jax.ShapeDtypeStruct((IMG, Q), jnp.float32),
            jax.ShapeDtypeStruct((NCLS, Q), jnp.float32),
            jax.ShapeDtypeStruct((1, Q), jnp.int32),
        ],
    )(img_t, w, probs_t)


def kernel(features, bank_features, bank_probs, image_bank):
    img_t = image_bank.transpose(1, 2, 3, 0).reshape(IMG, K)
    probs_t = bank_probs.transpose(1, 0)

    idx, w = _dist_topk(features, bank_features)
    grads = _sc_gather(idx.reshape(Q * KNN), bank_features)
    imgs_t, probs_t_out, labels = _select(img_t, w, probs_t)

    images = imgs_t.reshape(3, 32, 32, Q).transpose(3, 0, 1, 2)
    probs = probs_t_out.transpose(1, 0)
    return (labels.reshape(Q), probs, images, grads)


# speedup vs baseline: 5.5672x; 1.1363x over previous
import jax
import jax.numpy as jnp
from jax import lax
from jax.experimental import pallas as pl
from jax.experimental.pallas import tpu as pltpu
from jax.experimental.pallas import tpu_sc as plsc

Q = 256
D = 512
K = 16384
KNN = 4
NCLS = 10
IMG = 3072
KB = 4096
NSTEPS = K // KB
CHUNK = 2048
NCHUNK = K // CHUNK
KBS = 4096
NKS = K // KBS
MB = 1024
NMB = IMG // MB
EPS = 1e-12
BIGI = 2**30


def _dist_topk_body(f_ref, b_ref, idx_ref, w_ref, qn_ref, d_ref):
    step = pl.program_id(0)

    @pl.when(step == 0)
    def _():
        f = f_ref[:]
        nrm = jnp.sqrt(jnp.sum(f * f, axis=1, keepdims=True))
        qn_ref[:] = f / jnp.maximum(nrm, EPS)

    b = b_ref[:]
    nrm = jnp.sqrt(jnp.sum(b * b, axis=1, keepdims=True))
    kn = b / jnp.maximum(nrm, EPS)
    m = lax.dot_general(qn_ref[:], kn, (((1,), (1,)), ((), ())),
                        preferred_element_type=jnp.float32,
                        precision=lax.Precision.DEFAULT)
    dist = 1.0 - m
    for t in range(KB // CHUNK):
        d_ref[step * (KB // CHUNK) + t] = dist[:, t * CHUNK:(t + 1) * CHUNK]

    @pl.when(step == NSTEPS - 1)
    def _():
        found = []
        for j in range(KNN):
            def scan_chunk(c, carry):
                runmax, runidx = carry
                blk = d_ref[c]
                col = lax.broadcasted_iota(jnp.int32, (Q, CHUNK), 1) + c * CHUNK
                for f in found:
                    blk = jnp.where(col == f, -jnp.inf, blk)
                mc = jnp.max(blk, axis=1, keepdims=True)
                ac = jnp.min(jnp.where(blk == mc, col, BIGI),
                             axis=1, keepdims=True)
                take = mc > runmax
                return (jnp.where(take, mc, runmax),
                        jnp.where(take, ac, runidx))
            init = (jnp.full((Q, 1), -jnp.inf, jnp.float32),
                    jnp.full((Q, 1), BIGI, jnp.int32))
            _, best = lax.fori_loop(0, NCHUNK, scan_chunk, init)
            idx_ref[:, j:j + 1] = best
            found.append(best)
        for c in range(NCHUNK):
            col = lax.broadcasted_iota(jnp.int32, (Q, CHUNK), 1) + c * CHUNK
            w = jnp.zeros((Q, CHUNK), jnp.float32)
            for f in found:
                w = jnp.where(col == f, 0.25, w)
            w_ref[:, c * CHUNK:(c + 1) * CHUNK] = w.astype(jnp.bfloat16)


def _dist_topk(features, bank_features):
    return pl.pallas_call(
        _dist_topk_body,
        grid=(NSTEPS,),
        in_specs=[
            pl.BlockSpec((Q, D), lambda i: (0, 0)),
            pl.BlockSpec((KB, D), lambda i: (i, 0)),
        ],
        out_specs=[
            pl.BlockSpec((Q, KNN), lambda i: (0, 0)),
            pl.BlockSpec((Q, K), lambda i: (0, 0)),
        ],
        out_shape=[
            jax.ShapeDtypeStruct((Q, KNN), jnp.int32),
            jax.ShapeDtypeStruct((Q, K), jnp.bfloat16),
        ],
        scratch_shapes=[
            pltpu.VMEM((Q, D), jnp.float32),
            pltpu.VMEM((NCHUNK, Q, CHUNK), jnp.float32),
        ],
    )(features, bank_features)


QPW = Q // 32


def _sc_gather_body(idx_hbm, feat_hbm, grads_out, idx_v, feat_v, gsum_v, sem):
    wid = lax.axis_index("c") * 16 + lax.axis_index("s")
    qbase = wid * QPW

    pltpu.sync_copy(idx_hbm.at[pl.ds(qbase * KNN, QPW * KNN)], idx_v)
    pltpu.async_copy(feat_hbm.at[idx_v], feat_v, sem).wait()

    for q in range(QPW):
        def fbody(i, _, q=q):
            s = pl.ds(i * 16, 16)
            gsum_v[q, s] = 0.25 * ((feat_v[4 * q + 0, s] + feat_v[4 * q + 1, s])
                                   + (feat_v[4 * q + 2, s] + feat_v[4 * q + 3, s]))
            return 0
        lax.fori_loop(0, D // 16, fbody, 0)
    pltpu.sync_copy(gsum_v, grads_out.at[pl.ds(qbase, QPW)])


def _sc_gather(idx_flat, bank_features):
    mesh = plsc.VectorSubcoreMesh(core_axis_name="c", subcore_axis_name="s")
    fn = pl.kernel(
        _sc_gather_body,
        out_type=jax.ShapeDtypeStruct((Q, D), jnp.float32),
        mesh=mesh,
        scratch_types=[
            pltpu.VMEM((QPW * KNN,), jnp.int32),
            pltpu.VMEM((QPW * KNN, D), jnp.float32),
            pltpu.VMEM((QPW, D), jnp.float32),
            pltpu.SemaphoreType.DMA,
        ],
    )
    return fn(idx_flat, bank_features)


def _select_body(img_ref, w_ref, p_ref, imgs_ref, probs_ref, lab_ref):
    k = pl.program_id(0)
    m = pl.program_id(1)
    w = w_ref[:].astype(jnp.float32)
    blk = lax.dot_general(img_ref[:], w, (((1,), (1,)), ((), ())),
                          preferred_element_type=jnp.float32,
                          precision=lax.Precision.DEFAULT)
    sl = pl.ds(m * MB, MB)

    @pl.when(k == 0)
    def _():
        imgs_ref[sl, :] = blk

    @pl.when(k > 0)
    def _():
        imgs_ref[sl, :] += blk

    @pl.when(m == 0)
    def _():
        pblk = lax.dot_general(p_ref[:], w, (((1,), (1,)), ((), ())),
                               preferred_element_type=jnp.float32,
                               precision=lax.Precision.HIGHEST)

        @pl.when(k == 0)
        def _():
            probs_ref[:] = pblk

        @pl.when(k > 0)
        def _():
            probs_ref[:] += pblk

    @pl.when((m == NMB - 1) & (k == NKS - 1))
    def _():
        p = probs_ref[:]
        row = lax.broadcasted_iota(jnp.int32, (NCLS, Q), 0)
        top = jnp.max(p, axis=0, keepdims=True)
        lab_ref[:] = jnp.min(jnp.where(p == top, row, BIGI),
                             axis=0, keepdims=True)


def _select(img_t, w, probs_t):
    return pl.pallas_call(
        _select_body,
        grid=(NKS, NMB),
        in_specs=[
            pl.BlockSpec((MB, KBS), lambda k, m: (m, k)),
            pl.BlockSpec((Q, KBS), lambda k, m: (0, k)),
            pl.BlockSpec((NCLS, KBS), lambda k, m: (0, k)),
        ],
        out_specs=[
            pl.BlockSpec((IMG, Q), lambda k, m: (0, 0)),
            pl.BlockSpec((NCLS, Q), lambda k, m: (0, 0)),
            pl.BlockSpec((1, Q), lambda k, m: (0, 0)),
        ],
        out_shape=[
            jax.ShapeDtypeStruct((IMG, Q), jnp.float32),
            jax.ShapeDtypeStruct((NCLS, Q), jnp.float32),
            jax.ShapeDtypeStruct((1, Q), jnp.int32),
        ],
    )(img_t, w, probs_t)


def kernel(features, bank_features, bank_probs, image_bank):
    img_t = image_bank.transpose(1, 2, 3, 0).reshape(IMG, K)
    probs_t = bank_probs.transpose(1, 0)

    idx, w = _dist_topk(features, bank_features)
    grads = _sc_gather(idx.reshape(Q * KNN), bank_features)
    imgs_t, probs_t_out, labels = _select(img_t, w, probs_t)

    images = imgs_t.reshape(3, 32, 32, Q).transpose(3, 0, 1, 2)
    probs = probs_t_out.transpose(1, 0)
    return (labels.reshape(Q), probs, images, grads)
